# Initial kernel scaffold; baseline (speedup 1.0000x reference)
import jax
import jax.numpy as jnp
from jax.experimental import pallas as pl


def kernel(inputs, pos_embedding):
    raise NotImplementedError("write your pallas kernel here")


# speedup vs baseline: 2.8829x; 2.8829x over previous
import functools

import jax
import jax.numpy as jnp
from jax import lax
from jax.experimental import pallas as pl
from jax.experimental.pallas import tpu as pltpu
from jax.experimental.pallas import tpu_sc as plsc

_INFO = plsc.get_sparse_core_info()
_NC, _NS = _INFO.num_cores, _INFO.num_subcores
_NW = _NC * _NS

_CHUNK = 32


@functools.lru_cache(maxsize=None)
def _make_sc_broadcast(batch, seq_len, d_model):
    rows_per_w = seq_len // _NW
    assert rows_per_w * _NW == seq_len
    chunk = min(_CHUNK, rows_per_w)
    nchunk = rows_per_w // chunk
    assert nchunk * chunk == rows_per_w

    mesh = plsc.VectorSubcoreMesh(core_axis_name="c", subcore_axis_name="s")

    @functools.partial(
        pl.kernel,
        mesh=mesh,
        out_type=jax.ShapeDtypeStruct((batch, seq_len, d_model), jnp.float32),
        scratch_types=[pltpu.VMEM((chunk, d_model), jnp.float32)],
    )
    def sc_broadcast(table_hbm, out_hbm, buf):
        wid = lax.axis_index("s") * _NC + lax.axis_index("c")
        base = wid * rows_per_w

        def body(i, carry):
            row0 = base + i * chunk
            pltpu.sync_copy(table_hbm.at[pl.ds(row0, chunk)], buf)
            for b in range(batch):
                pltpu.sync_copy(buf, out_hbm.at[b, pl.ds(row0, chunk)])
            return carry

        lax.fori_loop(0, nchunk, body, 0)

    return sc_broadcast


def kernel(inputs, pos_embedding):
    batch, seq_len = inputs.shape
    d_model = pos_embedding.shape[1]
    return _make_sc_broadcast(batch, seq_len, d_model)(pos_embedding)


# speedup vs baseline: 2.9649x; 1.0285x over previous
import functools

import jax
import jax.numpy as jnp
from jax import lax
from jax.experimental import pallas as pl
from jax.experimental.pallas import tpu as pltpu
from jax.experimental.pallas import tpu_sc as plsc

_INFO = plsc.get_sparse_core_info()
_NC, _NS = _INFO.num_cores, _INFO.num_subcores
_NW = _NC * _NS

_CHUNK = 32


@functools.lru_cache(maxsize=None)
def _make_sc_broadcast(batch, seq_len, d_model):
    rows_per_w = seq_len // _NW
    assert rows_per_w * _NW == seq_len
    chunk = min(_CHUNK, rows_per_w)
    nchunk = rows_per_w // chunk
    assert nchunk * chunk == rows_per_w

    mesh = plsc.VectorSubcoreMesh(core_axis_name="c", subcore_axis_name="s")

    @functools.partial(
        pl.kernel,
        mesh=mesh,
        out_type=jax.ShapeDtypeStruct((batch, seq_len, d_model), jnp.float32),
        scratch_types=[
            pltpu.VMEM((2, chunk, d_model), jnp.float32),
            pltpu.SemaphoreType.DMA((2,)),
            pltpu.SemaphoreType.DMA((2, batch)),
        ],
    )
    def sc_broadcast(table_hbm, out_hbm, bufs, rsems, wsems):
        wid = lax.axis_index("s") * _NC + lax.axis_index("c")
        base = wid * rows_per_w

        def read(i, slot):
            return pltpu.make_async_copy(
                table_hbm.at[pl.ds(base + i * chunk, chunk)],
                bufs.at[slot],
                rsems.at[slot],
            )

        def write(i, slot, b):
            return pltpu.make_async_copy(
                bufs.at[slot],
                out_hbm.at[b, pl.ds(base + i * chunk, chunk)],
                wsems.at[slot, b],
            )

        read(0, 0).start()
        for i in range(nchunk):
            slot = i % 2
            read(i, slot).wait()
            if i + 1 < nchunk:
                nslot = (i + 1) % 2
                if i - 1 >= 0:
                    for b in range(batch):
                        write(i - 1, nslot, b).wait()
                read(i + 1, nslot).start()
            for b in range(batch):
                write(i, slot, b).start()
        for i in range(max(0, nchunk - 2), nchunk):
            for b in range(batch):
                write(i, i % 2, b).wait()

    return sc_broadcast


def kernel(inputs, pos_embedding):
    batch, seq_len = inputs.shape
    d_model = pos_embedding.shape[1]
    return _make_sc_broadcast(batch, seq_len, d_model)(pos_embedding)
